# baseline (speedup 1.0000x reference)
import functools

import jax
import jax.numpy as jnp
from jax import lax
from jax.experimental import pallas as pl
from jax.experimental.pallas import tpu as pltpu
from jax.experimental.pallas import tpu_sc as plsc

NUM_CORES = 2
NUM_SUBCORES = 16
NUM_WORKERS = NUM_CORES * NUM_SUBCORES
LANES = 16
NBUF = 4
SUBCHUNKS = 4
LEAD = 2


def _sc_body(x_hbm, p_hbm, o_hbm, xbuf, obuf, lsem, psem, ssem):
    B = x_hbm.shape[0]
    T = x_hbm.shape[1]
    D = x_hbm.shape[2]
    rows = T // NUM_WORKERS
    chunk = rows // SUBCHUNKS
    wid = lax.axis_index("s") * NUM_CORES + lax.axis_index("c")
    base = wid * rows

    nchunks = B * SUBCHUNKS

    def load(g, k):
        b, s = g // SUBCHUNKS, g % SUBCHUNKS
        return pltpu.make_async_copy(
            x_hbm.at[b, pl.ds(base + s * chunk, chunk)], xbuf.at[k], lsem.at[k])

    def pcopy(g, k):
        s = g % SUBCHUNKS
        return pltpu.make_async_copy(
            p_hbm.at[pl.ds(base + s * chunk, chunk)], obuf.at[k], psem.at[k])

    def store(g, k):
        b, s = g // SUBCHUNKS, g % SUBCHUNKS
        return pltpu.make_async_copy(
            obuf.at[k], o_hbm.at[b, pl.ds(base + s * chunk, chunk)], ssem.at[k])

    for k in range(NBUF):
        load(k, k).start()
    for k in range(LEAD):
        pcopy(k, k).start()

    @pl.loop(0, nchunks, step=NBUF)
    def _ring(g0):
        for j in range(NBUF):
            g = g0 + j
            load(g, j).wait()
            pcopy(g, j).wait()

            @plsc.parallel_loop(0, chunk, 1, unroll=4)
            def _add(r):
                for c in range(D // LANES):
                    sl = pl.ds(c * LANES, LANES)
                    plsc.addupdate(obuf.at[j, r, sl], xbuf[j, r, sl])

            store(g, j).start()

            @pl.when(g + NBUF < nchunks)
            def _():
                load(g + NBUF, j).start()

            @pl.when(g >= LEAD)
            def _():
                store(g - LEAD, (j - LEAD) % NBUF).wait()

            @pl.when(g + LEAD < nchunks)
            def _():
                pcopy(g + LEAD, (j + LEAD) % NBUF).start()

    for g in range(nchunks - LEAD, nchunks):
        store(g, g % NBUF).wait()


def kernel(x, pos_table):
    B, T, D = x.shape
    rows = T // NUM_WORKERS
    chunk = rows // SUBCHUNKS
    mesh = plsc.VectorSubcoreMesh(core_axis_name="c", subcore_axis_name="s")
    run = functools.partial(
        pl.kernel,
        mesh=mesh,
        out_type=jax.ShapeDtypeStruct((B, T, D), jnp.float32),
        scratch_types=[
            pltpu.VMEM((NBUF, chunk, D), jnp.float32),
            pltpu.VMEM((NBUF, chunk, D), jnp.float32),
            pltpu.SemaphoreType.DMA((NBUF,)),
            pltpu.SemaphoreType.DMA((NBUF,)),
            pltpu.SemaphoreType.DMA((NBUF,)),
        ],
    )(_sc_body)
    return run(x, pos_table)


# speedup vs baseline: 1.0768x; 1.0768x over previous
import functools

import jax
import jax.numpy as jnp
from jax import lax
from jax.experimental import pallas as pl
from jax.experimental.pallas import tpu as pltpu
from jax.experimental.pallas import tpu_sc as plsc

NUM_CORES = 2
NUM_SUBCORES = 16
NUM_WORKERS = NUM_CORES * NUM_SUBCORES
LANES = 16
NBUF = 2
SUBCHUNKS = 8


def _sc_body(x_hbm, p_hbm, o_hbm, p_v, xbuf, obuf, lsem, ssem):
    B = x_hbm.shape[0]
    rows, D = p_v.shape
    chunk = rows // SUBCHUNKS
    wid = lax.axis_index("s") * NUM_CORES + lax.axis_index("c")
    base = wid * rows

    def load(s, k, b):
        return pltpu.make_async_copy(
            x_hbm.at[b, pl.ds(base + s * chunk, chunk)],
            xbuf.at[k, b], lsem.at[k, b])

    def store(s, k, b):
        return pltpu.make_async_copy(
            obuf.at[k, b],
            o_hbm.at[b, pl.ds(base + s * chunk, chunk)], ssem.at[k, b])

    for k in range(NBUF):
        for b in range(B):
            load(k, k, b).start()

    pltpu.sync_copy(p_hbm.at[pl.ds(base, rows)], p_v)

    @pl.loop(0, SUBCHUNKS, step=NBUF)
    def _ring(s0):
        for j in range(NBUF):
            s = s0 + j
            for b in range(B):
                load(s, j, b).wait()

            @pl.when(s0 != 0)
            def _():
                for b in range(B):
                    store(s - NBUF, j, b).wait()

            off = s * chunk

            @plsc.parallel_loop(0, chunk, 1, unroll=2)
            def _add(r):
                for c in range(D // LANES):
                    sl = pl.ds(c * LANES, LANES)
                    pv = p_v[off + r, sl]
                    for b in range(B):
                        obuf[j, b, r, sl] = xbuf[j, b, r, sl] + pv

            for b in range(B):
                store(s, j, b).start()

            @pl.when(s + NBUF < SUBCHUNKS)
            def _():
                for b in range(B):
                    load(s + NBUF, j, b).start()

    for s in range(SUBCHUNKS - NBUF, SUBCHUNKS):
        for b in range(4):
            store(s, s % NBUF, b).wait()


def kernel(x, pos_table):
    B, T, D = x.shape
    rows = T // NUM_WORKERS
    chunk = rows // SUBCHUNKS
    mesh = plsc.VectorSubcoreMesh(core_axis_name="c", subcore_axis_name="s")
    run = functools.partial(
        pl.kernel,
        mesh=mesh,
        out_type=jax.ShapeDtypeStruct((B, T, D), jnp.float32),
        scratch_types=[
            pltpu.VMEM((rows, D), jnp.float32),
            pltpu.VMEM((NBUF, B, chunk, D), jnp.float32),
            pltpu.VMEM((NBUF, B, chunk, D), jnp.float32),
            pltpu.SemaphoreType.DMA((NBUF, B)),
            pltpu.SemaphoreType.DMA((NBUF, B)),
        ],
    )(_sc_body)
    return run(x, pos_table)


# speedup vs baseline: 1.1635x; 1.0805x over previous
import functools

import jax
import jax.numpy as jnp
from jax import lax
from jax.experimental import pallas as pl
from jax.experimental.pallas import tpu as pltpu
from jax.experimental.pallas import tpu_sc as plsc

NUM_CORES = 2
NUM_SUBCORES = 16
NUM_WORKERS = NUM_CORES * NUM_SUBCORES
LANES = 16
NBUF = 4
SUBCHUNKS = 4


def _sc_body(x_hbm, p_hbm, o_hbm, p_v, xbuf, obuf, lsem, ssem):
    B = x_hbm.shape[0]
    rows, D = p_v.shape
    chunk = rows // SUBCHUNKS
    wid = lax.axis_index("s") * NUM_CORES + lax.axis_index("c")
    base = wid * rows

    nchunks = B * SUBCHUNKS

    def load(g, k):
        b, s = g // SUBCHUNKS, g % SUBCHUNKS
        return pltpu.make_async_copy(
            x_hbm.at[b, pl.ds(base + s * chunk, chunk)], xbuf.at[k], lsem.at[k])

    def store(g, k):
        b, s = g // SUBCHUNKS, g % SUBCHUNKS
        return pltpu.make_async_copy(
            obuf.at[k], o_hbm.at[b, pl.ds(base + s * chunk, chunk)], ssem.at[k])

    for k in range(NBUF):
        load(k, k).start()

    pltpu.sync_copy(p_hbm.at[pl.ds(base, rows)], p_v)

    @pl.loop(0, nchunks, step=NBUF)
    def _ring(g0):
        for j in range(NBUF):
            g = g0 + j
            load(g, j).wait()

            @pl.when(g0 != 0)
            def _():
                store(g - NBUF, j).wait()

            off = (g % SUBCHUNKS) * chunk

            @plsc.parallel_loop(0, chunk, 1, unroll=4)
            def _add(r):
                for c in range(D // LANES):
                    sl = pl.ds(c * LANES, LANES)
                    obuf[j, r, sl] = xbuf[j, r, sl] + p_v[off + r, sl]

            store(g, j).start()

            @pl.when(g0 + NBUF < nchunks)
            def _():
                load(g + NBUF, j).start()

    for g in range(nchunks - NBUF, nchunks):
        store(g, g % NBUF).wait()


def kernel(x, pos_table):
    B, T, D = x.shape
    rows = T // NUM_WORKERS
    chunk = rows // SUBCHUNKS
    mesh = plsc.VectorSubcoreMesh(core_axis_name="c", subcore_axis_name="s")
    run = functools.partial(
        pl.kernel,
        mesh=mesh,
        out_type=jax.ShapeDtypeStruct((B, T, D), jnp.float32),
        scratch_types=[
            pltpu.VMEM((rows, D), jnp.float32),
            pltpu.VMEM((NBUF, chunk, D), jnp.float32),
            pltpu.VMEM((NBUF, chunk, D), jnp.float32),
            pltpu.SemaphoreType.DMA((NBUF,)),
            pltpu.SemaphoreType.DMA((NBUF,)),
        ],
    )(_sc_body)
    return run(x, pos_table)
